# baseline (speedup 1.0000x reference)
import functools

import jax
import jax.numpy as jnp
from jax import lax
from jax.experimental import pallas as pl
from jax.experimental.pallas import tpu as pltpu
from jax.experimental.pallas import tpu_sc as plsc

_BQ = 1024
_BB = 512
_NW = 32
_DL = 32


def _knn_body(x_ref, rt_ref, idx_ref, w_ref, r2_ref):
    x = x_ref[...]
    @pl.when(pl.program_id(0) == 0)
    def _():
        rt = rt_ref[...]
        r2_ref[...] = ((rt[0:1, :] * rt[0:1, :] + rt[2:3, :] * rt[2:3, :])
                       + rt[1:2, :] * rt[1:2, :]) + rt[3:4, :] * rt[3:4, :]

    q2 = (x[:, 0:1] * x[:, 0:1] + x[:, 2:3] * x[:, 2:3]) + x[:, 1:2] * x[:, 1:2]
    r2 = r2_ref[...]
    mm = jnp.dot(x, rt_ref[...], preferred_element_type=jnp.float32)
    d = q2 + r2 - 2.0 * mm
    bq, kp = d.shape
    iota = lax.broadcasted_iota(jnp.int32, (1, kp), 1).astype(jnp.float32)

    mins, args = [], []
    dcur = d
    for j in range(3):
        m = jnp.min(dcur, axis=1, keepdims=True)
        am = jnp.min(jnp.where(dcur == m, iota, jnp.float32(2**30)),
                     axis=1, keepdims=True)
        mins.append(m)
        args.append(am)
        if j < 2:
            dcur = jnp.where(iota == am, jnp.float32(1e30), dcur)

    dist = [jnp.sqrt(jnp.maximum(m, jnp.float32(1e-12))) for m in mins]
    d_sum = (dist[0] + dist[2]) + dist[1]
    dval = [d_sum - dj / d_sum for dj in dist]
    s = (dval[0] + dval[2]) + dval[1]

    idx_ref[...] = jnp.concatenate(args, axis=1).astype(jnp.int32)
    w_ref[...] = jnp.concatenate([dv / s for dv in dval], axis=1)


def _gather_body(idx_hbm, table_hbm, out_hbm, idx_v, rows_v, sem):
    n = idx_v.shape[0]
    wid = lax.axis_index("s") * 2 + lax.axis_index("c")
    base = wid * n
    pltpu.sync_copy(idx_hbm.at[pl.ds(base, n)], idx_v)
    pltpu.async_copy(table_hbm.at[idx_v], rows_v, sem).wait()
    pltpu.sync_copy(rows_v, out_hbm.at[pl.ds(base, n)])


def _blend_body(nn_ref, w_ref, out_ref):
    nn = nn_ref[...]
    w = w_ref[...]
    acc = (w[:, 0:1] * nn[:, 0 * _DL:1 * _DL]
           + w[:, 2:3] * nn[:, 2 * _DL:3 * _DL]) + w[:, 1:2] * nn[:, 1 * _DL:2 * _DL]
    out_ref[...] = acc


@jax.jit
def _run(scan_vertices, ref_vertices, smpl_lbs):
    q, _ = scan_vertices.shape
    k, nj = smpl_lbs.shape
    kp = ((k + 127) // 128) * 128
    qp = ((q + _BQ - 1) // _BQ) * _BQ
    q3 = qp * 3

    x = jnp.zeros((qp, 8), jnp.float32).at[:q, :3].set(scan_vertices)
    rt = (jnp.zeros((8, kp), jnp.float32)
          .at[:3, :k].set(ref_vertices.T)
          .at[3, k:].set(jnp.float32(1e15)))

    idx, w = pl.pallas_call(
        _knn_body,
        grid=(qp // _BQ,),
        in_specs=[
            pl.BlockSpec((_BQ, 8), lambda i: (i, 0)),
            pl.BlockSpec((8, kp), lambda i: (0, 0)),
        ],
        out_specs=[
            pl.BlockSpec((_BQ, 3), lambda i: (i, 0)),
            pl.BlockSpec((_BQ, 3), lambda i: (i, 0)),
        ],
        out_shape=[
            jax.ShapeDtypeStruct((qp, 3), jnp.int32),
            jax.ShapeDtypeStruct((qp, 3), jnp.float32),
        ],
        scratch_shapes=[pltpu.VMEM((1, kp), jnp.float32)],
        compiler_params=pltpu.CompilerParams(
            dimension_semantics=("arbitrary",),
        ),
    )(x, rt)

    table = jnp.zeros((k, _DL), jnp.float32).at[:, :nj].set(smpl_lbs)
    idx_flat = idx.reshape(q3)
    n_per_w = q3 // _NW

    gather = functools.partial(
        pl.kernel,
        out_type=jax.ShapeDtypeStruct((q3, _DL), jnp.float32),
        mesh=plsc.VectorSubcoreMesh(core_axis_name="c", subcore_axis_name="s"),
        scratch_types=[
            pltpu.VMEM((n_per_w,), jnp.int32),
            pltpu.VMEM((n_per_w, _DL), jnp.float32),
            pltpu.SemaphoreType.DMA,
        ],
        compiler_params=pltpu.CompilerParams(use_tc_tiling_on_sc=False),
    )(_gather_body)
    rows = gather(idx_flat, table)

    nn = rows.reshape(qp, 3 * _DL)
    out = pl.pallas_call(
        _blend_body,
        grid=(qp // _BB,),
        in_specs=[
            pl.BlockSpec((_BB, 3 * _DL), lambda i: (i, 0)),
            pl.BlockSpec((_BB, 3), lambda i: (i, 0)),
        ],
        out_specs=pl.BlockSpec((_BB, _DL), lambda i: (i, 0)),
        out_shape=jax.ShapeDtypeStruct((qp, _DL), jnp.float32),
        compiler_params=pltpu.CompilerParams(
            dimension_semantics=("arbitrary",),
        ),
    )(nn, w)
    return out[:q, :nj]


def kernel(scan_vertices, ref_vertices, smpl_lbs):
    return _run(scan_vertices, ref_vertices, smpl_lbs)


# speedup vs baseline: 1.0624x; 1.0624x over previous
import functools

import jax
import jax.numpy as jnp
from jax import lax
from jax.experimental import pallas as pl
from jax.experimental.pallas import tpu as pltpu
from jax.experimental.pallas import tpu_sc as plsc

_BQ = 512
_NW = 32
_DL = 32
_L = 16


def _knn_body(x_ref, rt_ref, idx_ref, w_ref, r2_ref):
    x = x_ref[...]
    @pl.when(pl.program_id(0) == 0)
    def _():
        rt = rt_ref[...]
        r2_ref[...] = ((rt[0:1, :] * rt[0:1, :] + rt[2:3, :] * rt[2:3, :])
                       + rt[1:2, :] * rt[1:2, :]) + rt[3:4, :] * rt[3:4, :]

    q2 = (x[:, 0:1] * x[:, 0:1] + x[:, 2:3] * x[:, 2:3]) + x[:, 1:2] * x[:, 1:2]
    r2 = r2_ref[...]
    mm = jnp.dot(x, rt_ref[...], preferred_element_type=jnp.float32)
    d = q2 + r2 - 2.0 * mm
    bq, kp = d.shape
    iota = lax.broadcasted_iota(jnp.int32, (1, kp), 1).astype(jnp.float32)

    mins, args = [], []
    dcur = d
    for j in range(3):
        m = jnp.min(dcur, axis=1, keepdims=True)
        am = jnp.min(jnp.where(dcur == m, iota, jnp.float32(2**30)),
                     axis=1, keepdims=True)
        mins.append(m)
        args.append(am)
        if j < 2:
            dcur = jnp.where(iota == am, jnp.float32(1e30), dcur)

    dist = [jnp.sqrt(jnp.maximum(m, jnp.float32(1e-12))) for m in mins]
    d_sum = (dist[0] + dist[2]) + dist[1]
    dval = [d_sum - dj / d_sum for dj in dist]
    s = (dval[0] + dval[2]) + dval[1]

    idx_ref[...] = jnp.concatenate(args, axis=1).astype(jnp.int32)
    w_ref[...] = jnp.concatenate([dv / s for dv in dval], axis=1)


def _gather_blend_body(idx_hbm, w_hbm, table_hbm, out_hbm,
                       idx_v, w_v, rows_v, out_v, sem):
    n3 = idx_v.shape[0]
    nq = n3 // 3
    wid = lax.axis_index("s") * 2 + lax.axis_index("c")
    base = wid * n3
    pltpu.sync_copy(idx_hbm.at[pl.ds(base, n3)], idx_v)
    pltpu.sync_copy(w_hbm.at[pl.ds(base, n3)], w_v)
    pltpu.async_copy(table_hbm.at[idx_v], rows_v, sem).wait()

    lane = lax.broadcasted_iota(jnp.int32, (_L,), 0)

    def body(i, carry):
        b3 = 3 * i
        w0 = plsc.load_gather(w_v, [jnp.broadcast_to(b3, (_L,))])
        w1 = plsc.load_gather(w_v, [jnp.broadcast_to(b3 + 1, (_L,))])
        w2 = plsc.load_gather(w_v, [jnp.broadcast_to(b3 + 2, (_L,))])
        for h in range(_DL // _L):
            col = lane + h * _L
            r0 = plsc.load_gather(rows_v, [jnp.broadcast_to(b3, (_L,)), col])
            r1 = plsc.load_gather(rows_v, [jnp.broadcast_to(b3 + 1, (_L,)), col])
            r2 = plsc.load_gather(rows_v, [jnp.broadcast_to(b3 + 2, (_L,)), col])
            out_v[pl.ds(_DL * i + h * _L, _L)] = (w0 * r0 + w2 * r2) + w1 * r1
        return carry

    lax.fori_loop(0, nq, body, 0)
    pltpu.sync_copy(out_v, out_hbm.at[pl.ds(wid * nq * _DL, nq * _DL)])


@jax.jit
def _run(scan_vertices, ref_vertices, smpl_lbs):
    q, _ = scan_vertices.shape
    k, nj = smpl_lbs.shape
    kp = ((k + 127) // 128) * 128
    qp = ((q + _BQ - 1) // _BQ) * _BQ
    q3 = qp * 3

    x = jnp.zeros((qp, 8), jnp.float32).at[:q, :3].set(scan_vertices)
    rt = (jnp.zeros((8, kp), jnp.float32)
          .at[:3, :k].set(ref_vertices.T)
          .at[3, k:].set(jnp.float32(1e15)))

    idx, w = pl.pallas_call(
        _knn_body,
        grid=(qp // _BQ,),
        in_specs=[
            pl.BlockSpec((_BQ, 8), lambda i: (i, 0)),
            pl.BlockSpec((8, kp), lambda i: (0, 0)),
        ],
        out_specs=[
            pl.BlockSpec((_BQ, 3), lambda i: (i, 0)),
            pl.BlockSpec((_BQ, 3), lambda i: (i, 0)),
        ],
        out_shape=[
            jax.ShapeDtypeStruct((qp, 3), jnp.int32),
            jax.ShapeDtypeStruct((qp, 3), jnp.float32),
        ],
        scratch_shapes=[pltpu.VMEM((1, kp), jnp.float32)],
        compiler_params=pltpu.CompilerParams(
            dimension_semantics=("arbitrary",),
        ),
    )(x, rt)

    table = jnp.zeros((k, _DL), jnp.float32).at[:, :nj].set(smpl_lbs)
    idx_flat = idx.reshape(q3)
    w_flat = w.reshape(q3)
    n_per_w = q3 // _NW
    nq_w = n_per_w // 3

    gather_blend = functools.partial(
        pl.kernel,
        out_type=jax.ShapeDtypeStruct((qp * _DL,), jnp.float32),
        mesh=plsc.VectorSubcoreMesh(core_axis_name="c", subcore_axis_name="s"),
        scratch_types=[
            pltpu.VMEM((n_per_w,), jnp.int32),
            pltpu.VMEM((n_per_w,), jnp.float32),
            pltpu.VMEM((n_per_w, _DL), jnp.float32),
            pltpu.VMEM((nq_w * _DL,), jnp.float32),
            pltpu.SemaphoreType.DMA,
        ],
        compiler_params=pltpu.CompilerParams(use_tc_tiling_on_sc=False,
                                             needs_layout_passes=False),
    )(_gather_blend_body)
    out = gather_blend(idx_flat, w_flat, table).reshape(qp, _DL)
    return out[:q, :nj]


def kernel(scan_vertices, ref_vertices, smpl_lbs):
    return _run(scan_vertices, ref_vertices, smpl_lbs)
